# baseline (speedup 1.0000x reference)
import functools

import jax
import jax.numpy as jnp
from jax import lax
from jax.experimental import pallas as pl
from jax.experimental.pallas import tpu as pltpu
from jax.experimental.pallas import tpu_sc as plsc

NC = 2
NS = 16
NW = NC * NS
L = 16


def _sc_mesh():
    return plsc.VectorSubcoreMesh(
        core_axis_name="c", subcore_axis_name="s", num_cores=NC, num_subcores=NS
    )


_SC_PARAMS = pltpu.CompilerParams(needs_layout_passes=False)


def _deg_call(ei, n_nodes):
    e = ei.shape[1]
    blocks = e // 128
    per = blocks // NW
    extra = blocks - per * NW
    w = per * 128

    @functools.partial(
        pl.kernel,
        out_type=[
            jax.ShapeDtypeStruct((NW, n_nodes), jnp.float32),
            jax.ShapeDtypeStruct((e,), jnp.int32),
            jax.ShapeDtypeStruct((e,), jnp.int32),
        ],
        mesh=_sc_mesh(),
        compiler_params=_SC_PARAMS,
        scratch_types=[
            pltpu.VMEM((2, w), jnp.int32),
            pltpu.VMEM((2, 128), jnp.int32),
            pltpu.VMEM((n_nodes,), jnp.float32),
            [pltpu.SemaphoreType.DMA for _ in range(3)],
        ],
    )
    def deg_kernel(ei_hbm, deg_hbm, src_hbm, dst_hbm, ebuf, xbuf, acc_v,
                   sems):
        cid = lax.axis_index("c")
        sid = lax.axis_index("s")
        wid = cid * NS + sid
        zeros16 = jnp.zeros((L,), jnp.float32)
        ones16 = jnp.ones((L,), jnp.float32)

        cp_main = pltpu.async_copy(ei_hbm.at[:, pl.ds(wid * w, w)], ebuf,
                                   sems[0])

        @pl.when(wid < extra)
        def _():
            pltpu.async_copy(
                ei_hbm.at[:, pl.ds((per * NW + wid) * 128, 128)], xbuf,
                sems[1])

        zu = 4
        nz = n_nodes // L

        def zero_body(i, _):
            for u in range(zu):
                acc_v[pl.ds(i * (zu * L) + u * L, L)] = zeros16
            return 0

        lax.fori_loop(0, nz // zu, zero_body, 0)
        for t in range((nz // zu) * zu, nz):
            acc_v[pl.ds(t * L, L)] = zeros16
        cp_main.wait()
        wbs = [
            pltpu.async_copy(ebuf.at[0], src_hbm.at[pl.ds(wid * w, w)],
                             sems[2]),
            pltpu.async_copy(ebuf.at[1], dst_hbm.at[pl.ds(wid * w, w)],
                             sems[2]),
        ]
        ku = 4
        nk = w // L

        def add16(k):
            idx = ebuf[1, pl.ds(k * L, L)]
            plsc.addupdate_scatter(acc_v, [idx], ones16)

        def add_body(i, _):
            for u in range(ku):
                add16(i * ku + u)
            return 0

        lax.fori_loop(0, nk // ku, add_body, 0)

        @pl.when(wid < extra)
        def _():
            pltpu.make_async_copy(ei_hbm.at[:, pl.ds(0, 128)], xbuf,
                                  sems[1]).wait()
            xbase = (per * NW) * 128 + wid * 128
            xwbs = [
                pltpu.async_copy(xbuf.at[0], src_hbm.at[pl.ds(xbase, 128)],
                                 sems[1]),
                pltpu.async_copy(xbuf.at[1], dst_hbm.at[pl.ds(xbase, 128)],
                                 sems[1]),
            ]
            for u in range(128 // L):
                idx = xbuf[1, pl.ds(u * L, L)]
                plsc.addupdate_scatter(acc_v, [idx], ones16)
            for d in xwbs:
                d.wait()

        for d in wbs:
            d.wait()
        pltpu.sync_copy(acc_v, deg_hbm.at[wid])

    return deg_kernel(ei)


def _tck1_call(x, w1, degp):
    n, d = x.shape
    h = w1.shape[1]

    def body(x_ref, w1_ref, degp_ref, hp_ref, dinv_col_ref, dinv_row_ref):
        degp = degp_ref[...]
        ones_col = jnp.ones((NW, 1), jnp.float32)
        deg_col = lax.dot_general(
            degp, ones_col, (((0,), (0,)), ((), ())),
            preferred_element_type=jnp.float32) + 1.0
        ones_row = jnp.ones((1, NW), jnp.float32)
        deg_row = lax.dot_general(
            ones_row, degp, (((1,), (0,)), ((), ())),
            preferred_element_type=jnp.float32) + 1.0
        dinv_col = lax.rsqrt(deg_col)
        dinv_row = lax.rsqrt(deg_row)
        hm = jnp.dot(x_ref[...].astype(jnp.bfloat16),
                     w1_ref[...].astype(jnp.bfloat16),
                     preferred_element_type=jnp.float32)
        hp_ref[...] = hm * dinv_col
        dinv_col_ref[...] = dinv_col
        dinv_row_ref[...] = dinv_row

    return pl.pallas_call(
        body,
        out_shape=[
            jax.ShapeDtypeStruct((n, h), jnp.float32),
            jax.ShapeDtypeStruct((n, 1), jnp.float32),
            jax.ShapeDtypeStruct((1, n), jnp.float32),
        ],
    )(x, w1, degp)


def _agg1_call(hp, src, dst):
    n, d = hp.shape
    e = src.shape[0]
    g = 80
    ept = e // NW
    nb = ept // g
    npad = -(-n // (8 * NS)) * (8 * NS)
    rpt = npad // NS

    nj = 8
    nr = 4

    @functools.partial(
        pl.kernel,
        out_type=jax.ShapeDtypeStruct((NC, npad, d), jnp.float32),
        mesh=_sc_mesh(),
        compiler_params=_SC_PARAMS,
        scratch_types=[
            [pltpu.VMEM((g,), jnp.int32) for _ in range(nj)],
            [pltpu.VMEM((g,), jnp.int32) for _ in range(nj)],
            [pltpu.VMEM((g, d), jnp.float32) for _ in range(nr)],
            [pltpu.SemaphoreType.DMA for _ in range(nj)],
            [pltpu.SemaphoreType.DMA for _ in range(nj)],
            [pltpu.SemaphoreType.DMA for _ in range(nr)],
            [pltpu.SemaphoreType.DMA for _ in range(nr)],
            pltpu.VMEM_SHARED((npad, d), jnp.float32),
        ],
    )
    def agg1_kernel(hp_hbm, src_hbm, dst_hbm, out_hbm,
                    sbufs, dbufs, rowbufs, isrc, idst, gsems, scsems, acc):
        cid = lax.axis_index("c")
        sid = lax.axis_index("s")
        zeros16 = jnp.zeros((L,), jnp.float32)

        def zrow(i, _):
            for k in range(d // L):
                rowbufs[0][i, pl.ds(k * L, L)] = zeros16
            return 0

        lax.fori_loop(0, g, zrow, 0)
        zcps = [pltpu.async_copy(rowbufs[0],
                                 acc.at[pl.ds(sid * rpt + t * g, g)],
                                 scsems[0])
                for t in range(rpt // g)]
        r0 = rpt - (rpt // g) * g
        if r0:
            zcps.append(pltpu.async_copy(
                rowbufs[0].at[pl.ds(0, r0)],
                acc.at[pl.ds(sid * rpt + (rpt // g) * g, r0)], scsems[0]))
        for cp in zcps:
            cp.wait()
        plsc.subcore_barrier()
        base0 = cid * (e // NC) + sid * ept

        def fetch_idx_slot(j, p):
            pltpu.async_copy(src_hbm.at[pl.ds(base0 + p * g, g)],
                             sbufs[j], isrc[j])
            pltpu.async_copy(dst_hbm.at[pl.ds(base0 + p * g, g)],
                             dbufs[j], idst[j])

        def wait_idx(j):
            pltpu.make_async_copy(src_hbm.at[pl.ds(0, g)],
                                  sbufs[j], isrc[j]).wait()
            pltpu.make_async_copy(dst_hbm.at[pl.ds(0, g)],
                                  dbufs[j], idst[j]).wait()

        def start_gather(j, r):
            pltpu.async_copy(hp_hbm.at[sbufs[j]], rowbufs[r], gsems[r])

        def wait_gather(r):
            pltpu.make_async_copy(hp_hbm.at[pl.ds(0, g)],
                                  rowbufs[r], gsems[r]).wait()

        def start_scatter(r, j):
            pltpu.async_copy(rowbufs[r], acc.at[dbufs[j]], scsems[r],
                             add=True)

        def drain_scatter(r, j):
            pltpu.make_async_copy(rowbufs[r], acc.at[dbufs[j]],
                                  scsems[r]).wait()

        def slot(p, pv):
            if p >= 4:
                drain_scatter((p - 4) % nr, (p - 4) % nj)
            if p + 4 < nb:
                fetch_idx_slot((p + 4) % nj, pv + 4)
            wait_idx(p % nj)
            start_gather(p % nj, p % nr)
            if p >= 2:
                wait_gather((p - 2) % nr)
                start_scatter((p - 2) % nr, (p - 2) % nj)

        for j in range(4):
            fetch_idx_slot(j, j)
        for p in range(8):
            slot(p, p)

        main_iters = nb // nj - 1
        rem = nb - (main_iters + 1) * nj

        def body(i, _):
            p0 = (i + 1) * nj
            for m in range(nj):
                drain_scatter(m % nr, (m + 4) % nj)
                fetch_idx_slot((m + 4) % nj, p0 + m + 4)
                wait_idx(m % nj)
                start_gather(m % nj, m % nr)
                wait_gather((m + 2) % nr)
                start_scatter((m + 2) % nr, (m + 6) % nj)
            return 0

        lax.fori_loop(0, main_iters, body, 0)
        for t in range(rem):
            p = (main_iters + 1) * nj + t
            slot(p, p)
        for p in range(nb, nb + 2):
            drain_scatter((p - 4) % nr, (p - 4) % nj)
            wait_gather((p - 2) % nr)
            start_scatter((p - 2) % nr, (p - 2) % nj)
        for p in range(nb + 2, nb + 4):
            drain_scatter((p - 4) % nr, (p - 4) % nj)
        plsc.subcore_barrier()
        pltpu.sync_copy(acc.at[pl.ds(sid * rpt, rpt)],
                        out_hbm.at[cid, pl.ds(sid * rpt, rpt)])

    return agg1_kernel(hp, src, dst)


def _tck2_call(accp, hp, dinv_col, dinv_row, b1, w2):
    n, h = hp.shape
    c = w2.shape[1]

    def body(accp_ref, hp_ref, dcol_ref, drow_ref, b1_ref, w2_ref, out_ref):
        s = accp_ref[0, :n] + accp_ref[1, :n] + hp_ref[...]
        out1 = s * dcol_ref[...] + b1_ref[...]
        r = jnp.maximum(out1, 0.0)
        h2t = lax.dot_general(
            w2_ref[...], r, (((0,), (1,)), ((), ())),
            preferred_element_type=jnp.float32)
        out_ref[...] = h2t * drow_ref[...]

    return pl.pallas_call(
        body,
        out_shape=jax.ShapeDtypeStruct((c, n), jnp.float32),
    )(accp, hp, dinv_col, dinv_row, b1, w2)


def _agg2_call(h2pt, src, dst):
    c, n = h2pt.shape
    e = src.shape[0]
    ept = e // NW

    @functools.partial(
        pl.kernel,
        out_type=jax.ShapeDtypeStruct((NW, c, n), jnp.float32),
        mesh=_sc_mesh(),
        compiler_params=_SC_PARAMS,
        scratch_types=[
            pltpu.VMEM((c, n), jnp.float32),
            pltpu.VMEM((ept,), jnp.int32),
            pltpu.VMEM((ept,), jnp.int32),
            pltpu.VMEM((c, n), jnp.float32),
            [pltpu.SemaphoreType.DMA for _ in range(3)],
        ],
    )
    def agg2_kernel(h2_hbm, src_hbm, dst_hbm, out_hbm, h2v, srcv, dstv, acc,
                    sems):
        cid = lax.axis_index("c")
        sid = lax.axis_index("s")
        wid = cid * NS + sid
        cps = [
            pltpu.async_copy(h2_hbm, h2v, sems[0]),
            pltpu.async_copy(src_hbm.at[pl.ds(wid * ept, ept)], srcv, sems[1]),
            pltpu.async_copy(dst_hbm.at[pl.ds(wid * ept, ept)], dstv, sems[2]),
        ]
        zeros16 = jnp.zeros((L,), jnp.float32)
        zu = 4
        nz = n // L

        def zero_body(i, _):
            for u in range(zu):
                for ch in range(c):
                    acc[ch, pl.ds(i * (zu * L) + u * L, L)] = zeros16
            return 0

        lax.fori_loop(0, nz // zu, zero_body, 0)
        for t in range((nz // zu) * zu, nz):
            for ch in range(c):
                acc[ch, pl.ds(t * L, L)] = zeros16
        for cp in cps:
            cp.wait()
        chan = [jnp.full((L,), ch, jnp.int32) for ch in range(c)]

        def edge16(k):
            s16 = srcv[pl.ds(k * L, L)]
            d16 = dstv[pl.ds(k * L, L)]
            for ch in range(c):
                v = plsc.load_gather(h2v, [chan[ch], s16])
                plsc.addupdate_scatter(acc, [chan[ch], d16], v)

        ku = 4
        nk = ept // L

        def body(i, _):
            for u in range(ku):
                edge16(i * ku + u)
            return 0

        lax.fori_loop(0, nk // ku, body, 0)
        for t in range((nk // ku) * ku, nk):
            edge16(t)
        pltpu.sync_copy(acc, out_hbm.at[wid])

    return agg2_kernel(h2pt, src, dst)


def _tck3_call(acc2p, h2pt, dinv_row, b2col):
    c, n = h2pt.shape

    def body(acc2p_ref, h2_ref, drow_ref, b2_ref, out_ref):
        acct = jnp.sum(acc2p_ref[...], axis=0)
        z = (acct + h2_ref[...]) * drow_ref[...] + b2_ref[...]
        m = jnp.max(z, axis=0, keepdims=True)
        lse = m + jnp.log(jnp.sum(jnp.exp(z - m), axis=0, keepdims=True))
        out_ref[...] = (z - lse).T

    return pl.pallas_call(
        body,
        out_shape=jax.ShapeDtypeStruct((n, c), jnp.float32),
    )(acc2p, h2pt, dinv_row, b2col)


def kernel(x, edge_index, W1, b1, W2, b2):
    n, d = x.shape
    c = W2.shape[1]
    degp, src, dst = _deg_call(edge_index, n)
    hp, dinv_col, dinv_row = _tck1_call(x, W1, degp)
    accp = _agg1_call(hp, src, dst)
    h2pt = _tck2_call(accp, hp, dinv_col, dinv_row, b1, W2)
    acc2p = _agg2_call(h2pt, src, dst)
    return _tck3_call(acc2p, h2pt, dinv_row, b2.reshape(c, 1))


# speedup vs baseline: 1.0375x; 1.0375x over previous
import functools

import jax
import jax.numpy as jnp
from jax import lax
from jax.experimental import pallas as pl
from jax.experimental.pallas import tpu as pltpu
from jax.experimental.pallas import tpu_sc as plsc

NC = 2
NS = 16
NW = NC * NS
L = 16


def _sc_mesh():
    return plsc.VectorSubcoreMesh(
        core_axis_name="c", subcore_axis_name="s", num_cores=NC, num_subcores=NS
    )


_SC_PARAMS = pltpu.CompilerParams(needs_layout_passes=False)


def _deg_call(ei, n_nodes):
    e = ei.shape[1]
    blocks = e // 128
    per = blocks // NW
    extra = blocks - per * NW
    w = per * 128

    @functools.partial(
        pl.kernel,
        out_type=[
            jax.ShapeDtypeStruct((NW, n_nodes), jnp.float32),
            jax.ShapeDtypeStruct((e,), jnp.int32),
            jax.ShapeDtypeStruct((e,), jnp.int32),
        ],
        mesh=_sc_mesh(),
        compiler_params=_SC_PARAMS,
        scratch_types=[
            pltpu.VMEM((2, w), jnp.int32),
            pltpu.VMEM((2, 128), jnp.int32),
            pltpu.VMEM((n_nodes,), jnp.float32),
            [pltpu.SemaphoreType.DMA for _ in range(3)],
        ],
    )
    def deg_kernel(ei_hbm, deg_hbm, src_hbm, dst_hbm, ebuf, xbuf, acc_v,
                   sems):
        cid = lax.axis_index("c")
        sid = lax.axis_index("s")
        wid = cid * NS + sid
        zeros16 = jnp.zeros((L,), jnp.float32)
        ones16 = jnp.ones((L,), jnp.float32)

        cp_main = pltpu.async_copy(ei_hbm.at[:, pl.ds(wid * w, w)], ebuf,
                                   sems[0])

        @pl.when(wid < extra)
        def _():
            pltpu.async_copy(
                ei_hbm.at[:, pl.ds((per * NW + wid) * 128, 128)], xbuf,
                sems[1])

        zu = 4
        nz = n_nodes // L

        def zero_body(i, _):
            for u in range(zu):
                acc_v[pl.ds(i * (zu * L) + u * L, L)] = zeros16
            return 0

        lax.fori_loop(0, nz // zu, zero_body, 0)
        for t in range((nz // zu) * zu, nz):
            acc_v[pl.ds(t * L, L)] = zeros16
        cp_main.wait()
        wbs = [
            pltpu.async_copy(ebuf.at[0], src_hbm.at[pl.ds(wid * w, w)],
                             sems[2]),
            pltpu.async_copy(ebuf.at[1], dst_hbm.at[pl.ds(wid * w, w)],
                             sems[2]),
        ]
        ku = 4
        nk = w // L

        def add16(k):
            idx = ebuf[1, pl.ds(k * L, L)]
            plsc.addupdate_scatter(acc_v, [idx], ones16)

        def add_body(i, _):
            for u in range(ku):
                add16(i * ku + u)
            return 0

        lax.fori_loop(0, nk // ku, add_body, 0)

        @pl.when(wid < extra)
        def _():
            pltpu.make_async_copy(ei_hbm.at[:, pl.ds(0, 128)], xbuf,
                                  sems[1]).wait()
            xbase = (per * NW) * 128 + wid * 128
            xwbs = [
                pltpu.async_copy(xbuf.at[0], src_hbm.at[pl.ds(xbase, 128)],
                                 sems[1]),
                pltpu.async_copy(xbuf.at[1], dst_hbm.at[pl.ds(xbase, 128)],
                                 sems[1]),
            ]
            for u in range(128 // L):
                idx = xbuf[1, pl.ds(u * L, L)]
                plsc.addupdate_scatter(acc_v, [idx], ones16)
            for d in xwbs:
                d.wait()

        for d in wbs:
            d.wait()
        pltpu.sync_copy(acc_v, deg_hbm.at[wid])

    return deg_kernel(ei)


def _tck1_call(x, w1, degp):
    n, d = x.shape
    h = w1.shape[1]

    def body(x_ref, w1_ref, degp_ref, hp_ref, dinv_col_ref, dinv_row_ref):
        degp = degp_ref[...]
        ones_col = jnp.ones((NW, 1), jnp.float32)
        deg_col = lax.dot_general(
            degp, ones_col, (((0,), (0,)), ((), ())),
            preferred_element_type=jnp.float32) + 1.0
        ones_row = jnp.ones((1, NW), jnp.float32)
        deg_row = lax.dot_general(
            ones_row, degp, (((1,), (0,)), ((), ())),
            preferred_element_type=jnp.float32) + 1.0
        dinv_col = lax.rsqrt(deg_col)
        dinv_row = lax.rsqrt(deg_row)
        hm = jnp.dot(x_ref[...].astype(jnp.bfloat16),
                     w1_ref[...].astype(jnp.bfloat16),
                     preferred_element_type=jnp.float32)
        hp_ref[...] = hm * dinv_col
        dinv_col_ref[...] = dinv_col
        dinv_row_ref[...] = dinv_row

    return pl.pallas_call(
        body,
        out_shape=[
            jax.ShapeDtypeStruct((n, h), jnp.float32),
            jax.ShapeDtypeStruct((n, 1), jnp.float32),
            jax.ShapeDtypeStruct((1, n), jnp.float32),
        ],
    )(x, w1, degp)


def _agg1_call(hp, src, dst):
    n, d = hp.shape
    e = src.shape[0]
    g = 80
    ept = e // NW
    nb = ept // g
    npad = -(-n // (8 * NS)) * (8 * NS)
    rpt = npad // NS

    nj = 8
    nr = 4

    @functools.partial(
        pl.kernel,
        out_type=jax.ShapeDtypeStruct((NC, npad, d), jnp.float32),
        mesh=_sc_mesh(),
        compiler_params=_SC_PARAMS,
        scratch_types=[
            [pltpu.VMEM((g,), jnp.int32) for _ in range(nj)],
            [pltpu.VMEM((g,), jnp.int32) for _ in range(nj)],
            [pltpu.VMEM((g, d), jnp.float32) for _ in range(nr)],
            [pltpu.SemaphoreType.DMA for _ in range(nj)],
            [pltpu.SemaphoreType.DMA for _ in range(nj)],
            [pltpu.SemaphoreType.DMA for _ in range(nr)],
            [pltpu.SemaphoreType.DMA for _ in range(nr)],
            pltpu.VMEM_SHARED((npad, d), jnp.float32),
        ],
    )
    def agg1_kernel(hp_hbm, src_hbm, dst_hbm, out_hbm,
                    sbufs, dbufs, rowbufs, isrc, idst, gsems, scsems, acc):
        cid = lax.axis_index("c")
        sid = lax.axis_index("s")
        zeros16 = jnp.zeros((L,), jnp.float32)

        def zrow(i, _):
            for k in range(d // L):
                rowbufs[0][i, pl.ds(k * L, L)] = zeros16
            return 0

        lax.fori_loop(0, g, zrow, 0)
        zcps = [pltpu.async_copy(rowbufs[0],
                                 acc.at[pl.ds(sid * rpt + t * g, g)],
                                 scsems[0])
                for t in range(rpt // g)]
        r0 = rpt - (rpt // g) * g
        if r0:
            zcps.append(pltpu.async_copy(
                rowbufs[0].at[pl.ds(0, r0)],
                acc.at[pl.ds(sid * rpt + (rpt // g) * g, r0)], scsems[0]))
        for cp in zcps:
            cp.wait()
        plsc.subcore_barrier()
        base0 = cid * (e // NC) + sid * ept

        def fetch_idx_slot(j, p):
            pltpu.async_copy(src_hbm.at[pl.ds(base0 + p * g, g)],
                             sbufs[j], isrc[j])
            pltpu.async_copy(dst_hbm.at[pl.ds(base0 + p * g, g)],
                             dbufs[j], idst[j])

        def wait_idx(j):
            pltpu.make_async_copy(src_hbm.at[pl.ds(0, g)],
                                  sbufs[j], isrc[j]).wait()
            pltpu.make_async_copy(dst_hbm.at[pl.ds(0, g)],
                                  dbufs[j], idst[j]).wait()

        def start_gather(j, r):
            pltpu.async_copy(hp_hbm.at[sbufs[j]], rowbufs[r], gsems[r])

        def wait_gather(r):
            pltpu.make_async_copy(hp_hbm.at[pl.ds(0, g)],
                                  rowbufs[r], gsems[r]).wait()

        def start_scatter(r, j):
            pltpu.async_copy(rowbufs[r], acc.at[dbufs[j]], scsems[r],
                             add=True)

        def drain_scatter(r, j):
            pltpu.make_async_copy(rowbufs[r], acc.at[dbufs[j]],
                                  scsems[r]).wait()

        def slot(p, pv):
            if p >= 4:
                drain_scatter((p - 4) % nr, (p - 4) % nj)
            if p + 4 < nb:
                fetch_idx_slot((p + 4) % nj, pv + 4)
            wait_idx(p % nj)
            start_gather(p % nj, p % nr)
            if p >= 2:
                wait_gather((p - 2) % nr)
                start_scatter((p - 2) % nr, (p - 2) % nj)

        for j in range(4):
            fetch_idx_slot(j, j)
        for p in range(8):
            slot(p, p)

        main_iters = nb // nj - 1
        rem = nb - (main_iters + 1) * nj

        def body(i, _):
            p0 = (i + 1) * nj
            for m in range(nj):
                drain_scatter(m % nr, (m + 4) % nj)
                fetch_idx_slot((m + 4) % nj, p0 + m + 4)
                wait_idx(m % nj)
                start_gather(m % nj, m % nr)
                wait_gather((m + 2) % nr)
                start_scatter((m + 2) % nr, (m + 6) % nj)
            return 0

        lax.fori_loop(0, main_iters, body, 0)
        for t in range(rem):
            p = (main_iters + 1) * nj + t
            slot(p, p)
        for p in range(nb, nb + 2):
            drain_scatter((p - 4) % nr, (p - 4) % nj)
            wait_gather((p - 2) % nr)
            start_scatter((p - 2) % nr, (p - 2) % nj)
        for p in range(nb + 2, nb + 4):
            drain_scatter((p - 4) % nr, (p - 4) % nj)
        plsc.subcore_barrier()
        pltpu.sync_copy(acc.at[pl.ds(sid * rpt, rpt)],
                        out_hbm.at[cid, pl.ds(sid * rpt, rpt)])

    return agg1_kernel(hp, src, dst)


def _tck2_call(accp, hp, dinv_col, dinv_row, b1, w2):
    n, h = hp.shape
    c = w2.shape[1]

    def body(accp_ref, hp_ref, dcol_ref, drow_ref, b1_ref, w2_ref, out_ref):
        s = accp_ref[0, :n] + accp_ref[1, :n] + hp_ref[...]
        out1 = s * dcol_ref[...] + b1_ref[...]
        r = jnp.maximum(out1, 0.0)
        h2t = lax.dot_general(
            w2_ref[...], r, (((0,), (1,)), ((), ())),
            preferred_element_type=jnp.float32)
        out_ref[...] = h2t * drow_ref[...]

    return pl.pallas_call(
        body,
        out_shape=jax.ShapeDtypeStruct((c, n), jnp.float32),
    )(accp, hp, dinv_col, dinv_row, b1, w2)


def _agg2_call(h2pt, src, dst):
    c, n = h2pt.shape
    e = src.shape[0]
    ept = e // NW

    @functools.partial(
        pl.kernel,
        out_type=jax.ShapeDtypeStruct((NW, c, n), jnp.float32),
        mesh=_sc_mesh(),
        compiler_params=_SC_PARAMS,
        scratch_types=[
            pltpu.VMEM((c, n), jnp.float32),
            pltpu.VMEM((ept,), jnp.int32),
            pltpu.VMEM((ept,), jnp.int32),
            pltpu.VMEM((c, n), jnp.float32),
            [pltpu.SemaphoreType.DMA for _ in range(3)],
        ],
    )
    def agg2_kernel(h2_hbm, src_hbm, dst_hbm, out_hbm, h2v, srcv, dstv, acc,
                    sems):
        cid = lax.axis_index("c")
        sid = lax.axis_index("s")
        wid = cid * NS + sid
        cps = [
            pltpu.async_copy(h2_hbm, h2v, sems[0]),
            pltpu.async_copy(src_hbm.at[pl.ds(wid * ept, ept)], srcv, sems[1]),
            pltpu.async_copy(dst_hbm.at[pl.ds(wid * ept, ept)], dstv, sems[2]),
        ]
        zeros16 = jnp.zeros((L,), jnp.float32)
        zu = 4
        nz = n // L

        def zero_body(i, _):
            for u in range(zu):
                for ch in range(c):
                    acc[ch, pl.ds(i * (zu * L) + u * L, L)] = zeros16
            return 0

        lax.fori_loop(0, nz // zu, zero_body, 0)
        for t in range((nz // zu) * zu, nz):
            for ch in range(c):
                acc[ch, pl.ds(t * L, L)] = zeros16
        for cp in cps:
            cp.wait()
        chan = [jnp.full((L,), ch, jnp.int32) for ch in range(c)]

        def edge16(k):
            s16 = srcv[pl.ds(k * L, L)]
            d16 = dstv[pl.ds(k * L, L)]
            for ch in range(c):
                v = plsc.load_gather(h2v, [chan[ch], s16])
                plsc.addupdate_scatter(acc, [chan[ch], d16], v)

        ku = 4
        nk = ept // L

        def body(i, _):
            for u in range(ku):
                edge16(i * ku + u)
            return 0

        lax.fori_loop(0, nk // ku, body, 0)
        for t in range((nk // ku) * ku, nk):
            edge16(t)
        pltpu.sync_copy(acc, out_hbm.at[wid])

    return agg2_kernel(h2pt, src, dst)


def _tck3_call(acc2p, h2pt, dinv_row, b2col):
    c, n = h2pt.shape

    def body(acc2p_ref, h2_ref, drow_ref, b2_ref, out_ref):
        acct = jnp.sum(acc2p_ref[...], axis=0)
        z = (acct + h2_ref[...]) * drow_ref[...] + b2_ref[...]
        m = jnp.max(z, axis=0, keepdims=True)
        lse = m + jnp.log(jnp.sum(jnp.exp(z - m), axis=0, keepdims=True))
        out_ref[...] = z - lse

    return pl.pallas_call(
        body,
        out_shape=jax.ShapeDtypeStruct((c, n), jnp.float32),
    )(acc2p, h2pt, dinv_row, b2col)


def kernel(x, edge_index, W1, b1, W2, b2):
    n, d = x.shape
    c = W2.shape[1]
    degp, src, dst = _deg_call(edge_index, n)
    hp, dinv_col, dinv_row = _tck1_call(x, W1, degp)
    accp = _agg1_call(hp, src, dst)
    h2pt = _tck2_call(accp, hp, dinv_col, dinv_row, b1, W2)
    acc2p = _agg2_call(h2pt, src, dst)
    outt = _tck3_call(acc2p, h2pt, dinv_row, b2.reshape(c, 1))
    return outt.T
